# baseline (speedup 1.0000x reference)
import jax
import jax.numpy as jnp
from jax import lax
from jax.experimental import pallas as pl
from jax.experimental.pallas import tpu as pltpu
from jax.experimental.pallas import tpu_sc as plsc

N_NODES = 10000
N_PAD = 10112
D_IN = 128
D_OUT = 128
D_EDGE = 16
D_DEG = 8
D_HALF = 64
N_EDGES = 320000
C = 128
NCHUNKS = N_EDGES // C
NC = 2
NS = 16
NW = NC * NS
NSLOTS = 158
RPT = N_PAD // NS
ZCHUNKS = (128, 128, 128, 128, 120)
ABK = 2560


def _attr_repack_body(attrT_ref, out_ref):
    out_ref[...] = attrT_ref[...].T


def _attr_repack(attrT):
    return pl.pallas_call(
        _attr_repack_body,
        grid=(N_EDGES // ABK,),
        in_specs=[pl.BlockSpec((D_EDGE, ABK), lambda i: (0, i))],
        out_specs=pl.BlockSpec((ABK, D_EDGE), lambda i: (i, 0)),
        out_shape=jax.ShapeDtypeStruct((N_EDGES, D_EDGE), jnp.float32),
    )(attrT)


def _sc_body(featsS_hbm, src_hbm, dst_hbm, attr_hbm, ones_hbm, zeros_hbm,
             out_node, out_attr, out_deg,
             sidx0, didx0, rows0, attrv0, sidx1, didx1, rows1, attrv1, ones_v,
             acc_node, acc_attr, acc_deg,
             sem_s0, sem_d0, sem_a0, sem_g0, sem_s1, sem_d1, sem_a1, sem_g1):
    cid = lax.axis_index("c")
    sid = lax.axis_index("s")
    zero16 = jnp.zeros((16,), jnp.float32)
    off16 = lax.broadcast(cid * N_NODES, (16,)).astype(jnp.int32)
    dump16 = jnp.full((16,), N_NODES, jnp.int32)

    def _fill_row(r, carry):
        for j in range(D_HALF // 16):
            rows0[r, pl.ds(j * 16, 16)] = zero16
        attrv0[r, :] = zero16
        return carry
    lax.fori_loop(0, C, _fill_row, 0)
    pltpu.sync_copy(ones_hbm, ones_v)

    r0 = sid * RPT
    zoff = 0
    for zc in ZCHUNKS:
        pltpu.sync_copy(rows0.at[pl.ds(0, zc)], acc_node.at[pl.ds(r0 + zoff, zc)])
        pltpu.sync_copy(attrv0.at[pl.ds(0, zc)], acc_attr.at[pl.ds(r0 + zoff, zc)])
        pltpu.sync_copy(zeros_hbm.at[pl.ds(0, zc)], acc_deg.at[pl.ds(r0 + zoff, zc)])
        zoff += zc
    plsc.subcore_barrier()

    def _base(slot):
        ch = sid + NS * slot
        return jnp.minimum(ch, NCHUNKS - 1) * C

    def _valid16(slot):
        v = (sid + NS * slot < NCHUNKS).astype(jnp.int32)
        return lax.broadcast(v, (16,))

    def _fix(sidx, didx, v16):
        for j in range(C // 16):
            sl = pl.ds(j * 16, 16)
            sidx[sl] = sidx[sl] + off16
            didx[sl] = didx[sl] * v16 + dump16 * (1 - v16)

    def _start_loads(slot, sidx, didx, attrv, ss, sd, sa):
        b = _base(slot)
        pltpu.async_copy(src_hbm.at[pl.ds(b, C)], sidx, ss)
        pltpu.async_copy(dst_hbm.at[pl.ds(b, C)], didx, sd)
        pltpu.async_copy(attr_hbm.at[pl.ds(b, C)], attrv, sa)

    def _wait_loads(sidx, didx, attrv, ss, sd, sa):
        pltpu.make_async_copy(src_hbm.at[pl.ds(0, C)], sidx, ss).wait()
        pltpu.make_async_copy(dst_hbm.at[pl.ds(0, C)], didx, sd).wait()
        pltpu.make_async_copy(attr_hbm.at[pl.ds(0, C)], attrv, sa).wait()

    def _scatter(rows, attrv, didx):
        pltpu.sync_copy(rows, acc_node.at[didx], add=True)
        pltpu.sync_copy(attrv, acc_attr.at[didx], add=True)
        pltpu.sync_copy(ones_v, acc_deg.at[didx], add=True)

    b0 = _base(0)
    pltpu.sync_copy(src_hbm.at[pl.ds(b0, C)], sidx0)
    pltpu.sync_copy(dst_hbm.at[pl.ds(b0, C)], didx0)
    pltpu.sync_copy(attr_hbm.at[pl.ds(b0, C)], attrv0)
    _fix(sidx0, didx0, _valid16(0))
    pltpu.async_copy(featsS_hbm.at[sidx0], rows0, sem_g0)
    _start_loads(1, sidx1, didx1, attrv1, sem_s1, sem_d1, sem_a1)

    def _pair(p, carry):
        pltpu.make_async_copy(featsS_hbm.at[sidx0], rows0, sem_g0).wait()
        _wait_loads(sidx1, didx1, attrv1, sem_s1, sem_d1, sem_a1)
        _fix(sidx1, didx1, _valid16(2 * p + 1))
        gb = pltpu.async_copy(featsS_hbm.at[sidx1], rows1, sem_g1)
        _scatter(rows0, attrv0, didx0)
        _start_loads(2 * p + 2, sidx0, didx0, attrv0, sem_s0, sem_d0, sem_a0)
        gb.wait()
        _scatter(rows1, attrv1, didx1)
        _start_loads(2 * p + 3, sidx1, didx1, attrv1, sem_s1, sem_d1, sem_a1)
        _wait_loads(sidx0, didx0, attrv0, sem_s0, sem_d0, sem_a0)
        _fix(sidx0, didx0, _valid16(2 * p + 2))
        pltpu.async_copy(featsS_hbm.at[sidx0], rows0, sem_g0)
        return carry
    lax.fori_loop(0, NSLOTS // 2, _pair, 0)

    pltpu.make_async_copy(featsS_hbm.at[sidx0], rows0, sem_g0).wait()
    pltpu.make_async_copy(src_hbm.at[pl.ds(0, C)], sidx1, sem_s1).wait()
    pltpu.make_async_copy(dst_hbm.at[pl.ds(0, C)], didx1, sem_d1).wait()
    pltpu.make_async_copy(attr_hbm.at[pl.ds(0, C)], attrv1, sem_a1).wait()
    plsc.subcore_barrier()

    pltpu.sync_copy(acc_node.at[pl.ds(r0, RPT)], out_node.at[cid, pl.ds(r0, RPT)])
    pltpu.sync_copy(acc_attr.at[pl.ds(r0, RPT)], out_attr.at[cid, pl.ds(r0, RPT)])
    pltpu.sync_copy(acc_deg.at[pl.ds(r0, RPT)], out_deg.at[cid, pl.ds(r0, RPT)])


def _sc_aggregate(featsS, src, dst, edge_attr, ones_d, zeros_d):
    mesh = plsc.VectorSubcoreMesh(core_axis_name="c", subcore_axis_name="s")
    kfn = pl.kernel(
        _sc_body,
        mesh=mesh,
        compiler_params=pltpu.CompilerParams(use_tc_tiling_on_sc=False),
        out_type=[
            jax.ShapeDtypeStruct((NC, N_PAD, D_HALF), jnp.float32),
            jax.ShapeDtypeStruct((NC, N_PAD, D_EDGE), jnp.float32),
            jax.ShapeDtypeStruct((NC, N_PAD, D_DEG), jnp.float32),
        ],
        scratch_types=[
            pltpu.VMEM((C,), jnp.int32),
            pltpu.VMEM((C,), jnp.int32),
            pltpu.VMEM((C, D_HALF), jnp.float32),
            pltpu.VMEM((C, D_EDGE), jnp.float32),
            pltpu.VMEM((C,), jnp.int32),
            pltpu.VMEM((C,), jnp.int32),
            pltpu.VMEM((C, D_HALF), jnp.float32),
            pltpu.VMEM((C, D_EDGE), jnp.float32),
            pltpu.VMEM((C, D_DEG), jnp.float32),
            pltpu.VMEM_SHARED((N_PAD, D_HALF), jnp.float32),
            pltpu.VMEM_SHARED((N_PAD, D_EDGE), jnp.float32),
            pltpu.VMEM_SHARED((N_PAD, D_DEG), jnp.float32),
            pltpu.SemaphoreType.DMA,
            pltpu.SemaphoreType.DMA,
            pltpu.SemaphoreType.DMA,
            pltpu.SemaphoreType.DMA,
            pltpu.SemaphoreType.DMA,
            pltpu.SemaphoreType.DMA,
            pltpu.SemaphoreType.DMA,
            pltpu.SemaphoreType.DMA,
        ],
    )
    return kfn(featsS, src, dst, edge_attr, ones_d, zeros_d)


def _combine_body(np_ref, ap_ref, dp_ref, feats_ref,
                  wrelt_ref, wedget_ref, wrest_ref,
                  bcomb_ref, bres_ref, gamma_ref, beta_ref, out_ref):
    aggf = jnp.concatenate(
        [np_ref[0, :N_NODES, :], np_ref[1, :N_NODES, :]], axis=1)
    segattr = ap_ref[0, :N_NODES, :]
    deg = dp_ref[1, :N_NODES, 0:1]
    agg = (jnp.dot(aggf, wrelt_ref[...], preferred_element_type=jnp.float32)
           + jnp.dot(segattr, wedget_ref[...], preferred_element_type=jnp.float32)
           + deg * bcomb_ref[...])
    new = jnp.maximum(agg, 0.0)
    res = jnp.maximum(
        jnp.dot(feats_ref[...], wrest_ref[...], preferred_element_type=jnp.float32)
        + bres_ref[...], 0.0)
    new = new + res
    mean = jnp.mean(new, axis=0, keepdims=True)
    var = jnp.mean((new - mean) ** 2, axis=0, keepdims=True)
    out_ref[...] = (new - mean) * lax.rsqrt(var + 1e-5) * gamma_ref[...] + beta_ref[...]


def _combine(node_p, attr_p, deg_p, feats, wrelt, wedget, wrest,
             bcomb, bres, gamma, beta):
    return pl.pallas_call(
        _combine_body,
        out_shape=jax.ShapeDtypeStruct((N_NODES, D_OUT), jnp.float32),
    )(node_p, attr_p, deg_p, feats, wrelt, wedget, wrest, bcomb, bres, gamma, beta)


def kernel(feats, edge_index, edge_attr, W_rel, b_rel, W_edge, b_edge,
           W_res, b_res, gamma, beta):
    src = edge_index[0]
    dst = edge_index[1]
    featsS = jnp.concatenate([feats[:, :D_HALF], feats[:, D_HALF:]], axis=0)
    attr_rm = _attr_repack(edge_attr.T)
    node_p, attr_p, deg_p = _sc_aggregate(
        featsS, src, dst, attr_rm,
        jnp.ones((C, D_DEG), jnp.float32), jnp.zeros((C, D_DEG), jnp.float32))
    return _combine(
        node_p, attr_p, deg_p, feats,
        W_rel.T, W_edge.T, W_res.T,
        (b_rel + b_edge).reshape(1, D_OUT), b_res.reshape(1, D_OUT),
        gamma.reshape(1, D_OUT), beta.reshape(1, D_OUT))


# speedup vs baseline: 1.4508x; 1.4508x over previous
import jax
import jax.numpy as jnp
from jax import lax
from jax.experimental import pallas as pl
from jax.experimental.pallas import tpu as pltpu
from jax.experimental.pallas import tpu_sc as plsc

N_NODES = 10000
N_PAD = 10112
D_IN = 128
D_OUT = 128
D_EDGE = 16
D_DEG = 8
D_HALF = 64
N_EDGES = 320000
C = 128
NCHUNKS = N_EDGES // C
NC = 2
NS = 16
NW = NC * NS
NSLOTS = 158
RPT = N_PAD // NS
ZCHUNKS = (128, 128, 128, 128, 120)
NSLOTS_A = 80


def _sc_node_body(featsS_hbm, src_hbm, dst_hbm,
                  out_node,
                  sidx0, didx0, rows0, sidx1, didx1, rows1,
                  acc_node,
                  sem_s0, sem_d0, sem_g0, sem_s1, sem_d1, sem_g1):
    cid = lax.axis_index("c")
    sid = lax.axis_index("s")
    zero16 = jnp.zeros((16,), jnp.float32)
    off16 = lax.broadcast(cid * N_NODES, (16,)).astype(jnp.int32)
    dump16 = jnp.full((16,), N_NODES, jnp.int32)

    def _fill_row(r, carry):
        for j in range(D_HALF // 16):
            rows0[r, pl.ds(j * 16, 16)] = zero16
        return carry
    lax.fori_loop(0, C, _fill_row, 0)

    r0 = sid * RPT
    zoff = 0
    for zc in ZCHUNKS:
        pltpu.sync_copy(rows0.at[pl.ds(0, zc)], acc_node.at[pl.ds(r0 + zoff, zc)])
        zoff += zc
    plsc.subcore_barrier()

    def _base(slot):
        return jnp.minimum(sid + NS * slot, NCHUNKS - 1) * C

    def _valid16(slot):
        v = (sid + NS * slot < NCHUNKS).astype(jnp.int32)
        return lax.broadcast(v, (16,))

    def _fix(sidx, didx, v16):
        for j in range(C // 16):
            sl = pl.ds(j * 16, 16)
            sidx[sl] = sidx[sl] + off16
            didx[sl] = didx[sl] * v16 + dump16 * (1 - v16)

    def _start_loads(slot, sidx, didx, ss, sd):
        b = _base(slot)
        pltpu.async_copy(src_hbm.at[pl.ds(b, C)], sidx, ss)
        pltpu.async_copy(dst_hbm.at[pl.ds(b, C)], didx, sd)

    def _wait_loads(sidx, didx, ss, sd):
        pltpu.make_async_copy(src_hbm.at[pl.ds(0, C)], sidx, ss).wait()
        pltpu.make_async_copy(dst_hbm.at[pl.ds(0, C)], didx, sd).wait()

    b0 = _base(0)
    pltpu.sync_copy(src_hbm.at[pl.ds(b0, C)], sidx0)
    pltpu.sync_copy(dst_hbm.at[pl.ds(b0, C)], didx0)
    _fix(sidx0, didx0, _valid16(0))
    pltpu.async_copy(featsS_hbm.at[sidx0], rows0, sem_g0)
    _start_loads(1, sidx1, didx1, sem_s1, sem_d1)

    def _pair(p, carry):
        pltpu.make_async_copy(featsS_hbm.at[sidx0], rows0, sem_g0).wait()
        _wait_loads(sidx1, didx1, sem_s1, sem_d1)
        _fix(sidx1, didx1, _valid16(2 * p + 1))
        gb = pltpu.async_copy(featsS_hbm.at[sidx1], rows1, sem_g1)
        pltpu.sync_copy(rows0, acc_node.at[didx0], add=True)
        _start_loads(2 * p + 2, sidx0, didx0, sem_s0, sem_d0)
        gb.wait()
        pltpu.sync_copy(rows1, acc_node.at[didx1], add=True)
        _start_loads(2 * p + 3, sidx1, didx1, sem_s1, sem_d1)
        _wait_loads(sidx0, didx0, sem_s0, sem_d0)
        _fix(sidx0, didx0, _valid16(2 * p + 2))
        pltpu.async_copy(featsS_hbm.at[sidx0], rows0, sem_g0)
        return carry
    lax.fori_loop(0, NSLOTS // 2, _pair, 0)

    pltpu.make_async_copy(featsS_hbm.at[sidx0], rows0, sem_g0).wait()
    pltpu.make_async_copy(src_hbm.at[pl.ds(0, C)], sidx1, sem_s1).wait()
    pltpu.make_async_copy(dst_hbm.at[pl.ds(0, C)], didx1, sem_d1).wait()
    plsc.subcore_barrier()

    pltpu.sync_copy(acc_node.at[pl.ds(r0, RPT)], out_node.at[cid, pl.ds(r0, RPT)])


def _sc_attr_body(dst_hbm, attr_hbm, ones_hbm, zeros_hbm,
                  out_attr, out_deg,
                  didx0, attrv0, didx1, attrv1, ones_v,
                  acc_attr, acc_deg,
                  sem_d0, sem_a0, sem_d1, sem_a1):
    cid = lax.axis_index("c")
    sid = lax.axis_index("s")
    wid = sid * NC + cid
    zero16 = jnp.zeros((16,), jnp.float32)
    dump16 = jnp.full((16,), N_NODES, jnp.int32)

    def _fill_row(r, carry):
        attrv0[r, :] = zero16
        return carry
    lax.fori_loop(0, C, _fill_row, 0)
    pltpu.sync_copy(ones_hbm, ones_v)

    r0 = sid * RPT
    zoff = 0
    for zc in ZCHUNKS:
        pltpu.sync_copy(attrv0.at[pl.ds(0, zc)], acc_attr.at[pl.ds(r0 + zoff, zc)])
        pltpu.sync_copy(zeros_hbm.at[pl.ds(0, zc)], acc_deg.at[pl.ds(r0 + zoff, zc)])
        zoff += zc
    plsc.subcore_barrier()

    def _base(slot):
        return jnp.minimum(wid + NW * slot, NCHUNKS - 1) * C

    def _valid16(slot):
        v = (wid + NW * slot < NCHUNKS).astype(jnp.int32)
        return lax.broadcast(v, (16,))

    def _mask(didx, v16):
        for j in range(C // 16):
            sl = pl.ds(j * 16, 16)
            didx[sl] = didx[sl] * v16 + dump16 * (1 - v16)

    def _start_loads(slot, didx, attrv, sd, sa):
        b = _base(slot)
        pltpu.async_copy(dst_hbm.at[pl.ds(b, C)], didx, sd)
        pltpu.async_copy(attr_hbm.at[pl.ds(b, C)], attrv, sa)

    def _wait_loads(didx, attrv, sd, sa):
        pltpu.make_async_copy(dst_hbm.at[pl.ds(0, C)], didx, sd).wait()
        pltpu.make_async_copy(attr_hbm.at[pl.ds(0, C)], attrv, sa).wait()

    b0 = _base(0)
    pltpu.sync_copy(dst_hbm.at[pl.ds(b0, C)], didx0)
    pltpu.sync_copy(attr_hbm.at[pl.ds(b0, C)], attrv0)
    _mask(didx0, _valid16(0))
    _start_loads(1, didx1, attrv1, sem_d1, sem_a1)

    def _pair(p, carry):
        pltpu.sync_copy(attrv0, acc_attr.at[didx0], add=True)
        pltpu.sync_copy(ones_v, acc_deg.at[didx0], add=True)
        _wait_loads(didx1, attrv1, sem_d1, sem_a1)
        _mask(didx1, _valid16(2 * p + 1))
        _start_loads(2 * p + 2, didx0, attrv0, sem_d0, sem_a0)
        pltpu.sync_copy(attrv1, acc_attr.at[didx1], add=True)
        pltpu.sync_copy(ones_v, acc_deg.at[didx1], add=True)
        _start_loads(2 * p + 3, didx1, attrv1, sem_d1, sem_a1)
        _wait_loads(didx0, attrv0, sem_d0, sem_a0)
        _mask(didx0, _valid16(2 * p + 2))
        return carry
    lax.fori_loop(0, NSLOTS_A // 2, _pair, 0)

    pltpu.make_async_copy(dst_hbm.at[pl.ds(0, C)], didx1, sem_d1).wait()
    pltpu.make_async_copy(attr_hbm.at[pl.ds(0, C)], attrv1, sem_a1).wait()
    plsc.subcore_barrier()

    pltpu.sync_copy(acc_attr.at[pl.ds(r0, RPT)], out_attr.at[cid, pl.ds(r0, RPT)])
    pltpu.sync_copy(acc_deg.at[pl.ds(r0, RPT)], out_deg.at[cid, pl.ds(r0, RPT)])


def _sc_aggregate(featsS, src, dst, edge_attr, ones_d, zeros_d):
    mesh = plsc.VectorSubcoreMesh(core_axis_name="c", subcore_axis_name="s")
    node_fn = pl.kernel(
        _sc_node_body,
        mesh=mesh,
        compiler_params=pltpu.CompilerParams(use_tc_tiling_on_sc=False),
        out_type=[jax.ShapeDtypeStruct((NC, N_PAD, D_HALF), jnp.float32)],
        scratch_types=[
            pltpu.VMEM((C,), jnp.int32),
            pltpu.VMEM((C,), jnp.int32),
            pltpu.VMEM((C, D_HALF), jnp.float32),
            pltpu.VMEM((C,), jnp.int32),
            pltpu.VMEM((C,), jnp.int32),
            pltpu.VMEM((C, D_HALF), jnp.float32),
            pltpu.VMEM_SHARED((N_PAD, D_HALF), jnp.float32),
            pltpu.SemaphoreType.DMA,
            pltpu.SemaphoreType.DMA,
            pltpu.SemaphoreType.DMA,
            pltpu.SemaphoreType.DMA,
            pltpu.SemaphoreType.DMA,
            pltpu.SemaphoreType.DMA,
        ],
    )
    attr_fn = pl.kernel(
        _sc_attr_body,
        mesh=mesh,
        compiler_params=pltpu.CompilerParams(use_tc_tiling_on_sc=False),
        out_type=[
            jax.ShapeDtypeStruct((NC, N_PAD, D_EDGE), jnp.float32),
            jax.ShapeDtypeStruct((NC, N_PAD, D_DEG), jnp.float32),
        ],
        scratch_types=[
            pltpu.VMEM((C,), jnp.int32),
            pltpu.VMEM((C, D_EDGE), jnp.float32),
            pltpu.VMEM((C,), jnp.int32),
            pltpu.VMEM((C, D_EDGE), jnp.float32),
            pltpu.VMEM((C, D_DEG), jnp.float32),
            pltpu.VMEM_SHARED((N_PAD, D_EDGE), jnp.float32),
            pltpu.VMEM_SHARED((N_PAD, D_DEG), jnp.float32),
            pltpu.SemaphoreType.DMA,
            pltpu.SemaphoreType.DMA,
            pltpu.SemaphoreType.DMA,
            pltpu.SemaphoreType.DMA,
        ],
    )
    (node_p,) = node_fn(featsS, src, dst)
    attr_p, deg_p = attr_fn(dst, edge_attr, ones_d, zeros_d)
    return node_p, attr_p, deg_p


def _combine_body(np_ref, ap_ref, dp_ref, feats_ref,
                  wrelt_ref, wedget_ref, wrest_ref,
                  bcomb_ref, bres_ref, gamma_ref, beta_ref, out_ref):
    aggf = jnp.concatenate(
        [np_ref[0, :N_NODES, :], np_ref[1, :N_NODES, :]], axis=1)
    segattr = ap_ref[0, :N_NODES, :] + ap_ref[1, :N_NODES, :]
    deg = dp_ref[0, :N_NODES, 0:1] + dp_ref[1, :N_NODES, 0:1]
    agg = (jnp.dot(aggf, wrelt_ref[...], preferred_element_type=jnp.float32)
           + jnp.dot(segattr, wedget_ref[...], preferred_element_type=jnp.float32)
           + deg * bcomb_ref[...])
    new = jnp.maximum(agg, 0.0)
    res = jnp.maximum(
        jnp.dot(feats_ref[...], wrest_ref[...], preferred_element_type=jnp.float32)
        + bres_ref[...], 0.0)
    new = new + res
    mean = jnp.mean(new, axis=0, keepdims=True)
    var = jnp.mean((new - mean) ** 2, axis=0, keepdims=True)
    out_ref[...] = (new - mean) * lax.rsqrt(var + 1e-5) * gamma_ref[...] + beta_ref[...]


def _combine(node_p, attr_p, deg_p, feats, wrelt, wedget, wrest,
             bcomb, bres, gamma, beta):
    return pl.pallas_call(
        _combine_body,
        out_shape=jax.ShapeDtypeStruct((N_NODES, D_OUT), jnp.float32),
    )(node_p, attr_p, deg_p, feats, wrelt, wedget, wrest, bcomb, bres, gamma, beta)


def kernel(feats, edge_index, edge_attr, W_rel, b_rel, W_edge, b_edge,
           W_res, b_res, gamma, beta):
    src = edge_index[0]
    dst = edge_index[1]
    featsS = jnp.concatenate([feats[:, :D_HALF], feats[:, D_HALF:]], axis=0)
    node_p, attr_p, deg_p = _sc_aggregate(
        featsS, src, dst, edge_attr,
        jnp.ones((C, D_DEG), jnp.float32), jnp.zeros((C, D_DEG), jnp.float32))
    return _combine(
        node_p, attr_p, deg_p, feats,
        W_rel.T, W_edge.T, W_res.T,
        (b_rel + b_edge).reshape(1, D_OUT), b_res.reshape(1, D_OUT),
        gamma.reshape(1, D_OUT), beta.reshape(1, D_OUT))


# speedup vs baseline: 1.6664x; 1.1486x over previous
import jax
import jax.numpy as jnp
from jax import lax
from jax.experimental import pallas as pl
from jax.experimental.pallas import tpu as pltpu
from jax.experimental.pallas import tpu_sc as plsc

N_NODES = 10000
N_PAD = 10112
D_IN = 128
D_OUT = 128
D_EDGE = 16
D_DEG = 8
D_HALF = 64
N_EDGES = 320000
C = 128
NCHUNKS = N_EDGES // C
NC = 2
NS = 16
NW = NC * NS
NSLOTS = 158
RPT = N_PAD // NS
ZCHUNKS = (128, 128, 128, 128, 120)
NSLOTS_A = 80


def _sc_node_body(featsS_hbm, src_hbm, dst_hbm,
                  out_node,
                  sidx0, didx0, rows0, sidx1, didx1, rows1,
                  acc_node,
                  sem_s0, sem_d0, sem_g0, sem_s1, sem_d1, sem_g1):
    cid = lax.axis_index("c")
    sid = lax.axis_index("s")
    zero16 = jnp.zeros((16,), jnp.float32)
    cid16 = lax.broadcast(cid, (16,)).astype(jnp.int32)
    dump16 = jnp.full((16,), N_NODES, jnp.int32)

    def _fill_row(r, carry):
        for j in range(D_HALF // 16):
            rows0[r, pl.ds(j * 16, 16)] = zero16
        return carry
    lax.fori_loop(0, C, _fill_row, 0)

    r0 = sid * RPT
    zoff = 0
    for zc in ZCHUNKS:
        pltpu.sync_copy(rows0.at[pl.ds(0, zc)], acc_node.at[pl.ds(r0 + zoff, zc)])
        zoff += zc
    plsc.subcore_barrier()

    def _base(slot):
        return jnp.minimum(sid + NS * slot, NCHUNKS - 1) * C

    def _valid16(slot):
        v = (sid + NS * slot < NCHUNKS).astype(jnp.int32)
        return lax.broadcast(v, (16,))

    def _fix(sidx, didx, v16):
        for j in range(C // 16):
            sl = pl.ds(j * 16, 16)
            sidx[sl] = sidx[sl] * 2 + cid16
            didx[sl] = didx[sl] * v16 + dump16 * (1 - v16)

    def _start_loads(slot, sidx, didx, ss, sd):
        b = _base(slot)
        pltpu.async_copy(src_hbm.at[pl.ds(b, C)], sidx, ss)
        pltpu.async_copy(dst_hbm.at[pl.ds(b, C)], didx, sd)

    def _wait_loads(sidx, didx, ss, sd):
        pltpu.make_async_copy(src_hbm.at[pl.ds(0, C)], sidx, ss).wait()
        pltpu.make_async_copy(dst_hbm.at[pl.ds(0, C)], didx, sd).wait()

    for slot, sidx, didx, rows, sg in (
        (0, sidx0, didx0, rows0, sem_g0),
        (1, sidx1, didx1, rows1, sem_g1),
    ):
        b = _base(slot)
        pltpu.sync_copy(src_hbm.at[pl.ds(b, C)], sidx)
        pltpu.sync_copy(dst_hbm.at[pl.ds(b, C)], didx)
        _fix(sidx, didx, _valid16(slot))
        pltpu.async_copy(featsS_hbm.at[sidx], rows, sg)

    def _pair(p, carry):
        pltpu.make_async_copy(featsS_hbm.at[sidx0], rows0, sem_g0).wait()
        pltpu.sync_copy(rows0, acc_node.at[didx0], add=True)
        _start_loads(2 * p + 2, sidx0, didx0, sem_s0, sem_d0)
        pltpu.make_async_copy(featsS_hbm.at[sidx1], rows1, sem_g1).wait()
        pltpu.sync_copy(rows1, acc_node.at[didx1], add=True)
        _start_loads(2 * p + 3, sidx1, didx1, sem_s1, sem_d1)
        _wait_loads(sidx0, didx0, sem_s0, sem_d0)
        _fix(sidx0, didx0, _valid16(2 * p + 2))
        pltpu.async_copy(featsS_hbm.at[sidx0], rows0, sem_g0)
        _wait_loads(sidx1, didx1, sem_s1, sem_d1)
        _fix(sidx1, didx1, _valid16(2 * p + 3))
        pltpu.async_copy(featsS_hbm.at[sidx1], rows1, sem_g1)
        return carry
    lax.fori_loop(0, NSLOTS // 2, _pair, 0)

    pltpu.make_async_copy(featsS_hbm.at[sidx0], rows0, sem_g0).wait()
    pltpu.make_async_copy(featsS_hbm.at[sidx1], rows1, sem_g1).wait()
    plsc.subcore_barrier()

    pltpu.sync_copy(acc_node.at[pl.ds(r0, RPT)], out_node.at[cid, pl.ds(r0, RPT)])


def _sc_attr_body(dst_hbm, attr_hbm, ones_hbm, zeros_hbm,
                  out_attr, out_deg,
                  didx0, attrv0, didx1, attrv1, ones_v,
                  acc_attr, acc_deg,
                  sem_d0, sem_a0, sem_d1, sem_a1):
    cid = lax.axis_index("c")
    sid = lax.axis_index("s")
    wid = sid * NC + cid
    zero16 = jnp.zeros((16,), jnp.float32)
    dump16 = jnp.full((16,), N_NODES, jnp.int32)

    def _fill_row(r, carry):
        attrv0[r, :] = zero16
        return carry
    lax.fori_loop(0, C, _fill_row, 0)
    pltpu.sync_copy(ones_hbm, ones_v)

    r0 = sid * RPT
    zoff = 0
    for zc in ZCHUNKS:
        pltpu.sync_copy(attrv0.at[pl.ds(0, zc)], acc_attr.at[pl.ds(r0 + zoff, zc)])
        pltpu.sync_copy(zeros_hbm.at[pl.ds(0, zc)], acc_deg.at[pl.ds(r0 + zoff, zc)])
        zoff += zc
    plsc.subcore_barrier()

    def _base(slot):
        return jnp.minimum(wid + NW * slot, NCHUNKS - 1) * C

    def _valid16(slot):
        v = (wid + NW * slot < NCHUNKS).astype(jnp.int32)
        return lax.broadcast(v, (16,))

    def _mask(didx, v16):
        for j in range(C // 16):
            sl = pl.ds(j * 16, 16)
            didx[sl] = didx[sl] * v16 + dump16 * (1 - v16)

    def _start_loads(slot, didx, attrv, sd, sa):
        b = _base(slot)
        pltpu.async_copy(dst_hbm.at[pl.ds(b, C)], didx, sd)
        pltpu.async_copy(attr_hbm.at[pl.ds(b, C)], attrv, sa)

    def _wait_loads(didx, attrv, sd, sa):
        pltpu.make_async_copy(dst_hbm.at[pl.ds(0, C)], didx, sd).wait()
        pltpu.make_async_copy(attr_hbm.at[pl.ds(0, C)], attrv, sa).wait()

    b0 = _base(0)
    pltpu.sync_copy(dst_hbm.at[pl.ds(b0, C)], didx0)
    pltpu.sync_copy(attr_hbm.at[pl.ds(b0, C)], attrv0)
    _mask(didx0, _valid16(0))
    _start_loads(1, didx1, attrv1, sem_d1, sem_a1)

    def _pair(p, carry):
        pltpu.sync_copy(attrv0, acc_attr.at[didx0], add=True)
        pltpu.sync_copy(ones_v, acc_deg.at[didx0], add=True)
        _wait_loads(didx1, attrv1, sem_d1, sem_a1)
        _mask(didx1, _valid16(2 * p + 1))
        _start_loads(2 * p + 2, didx0, attrv0, sem_d0, sem_a0)
        pltpu.sync_copy(attrv1, acc_attr.at[didx1], add=True)
        pltpu.sync_copy(ones_v, acc_deg.at[didx1], add=True)
        _start_loads(2 * p + 3, didx1, attrv1, sem_d1, sem_a1)
        _wait_loads(didx0, attrv0, sem_d0, sem_a0)
        _mask(didx0, _valid16(2 * p + 2))
        return carry
    lax.fori_loop(0, NSLOTS_A // 2, _pair, 0)

    pltpu.make_async_copy(dst_hbm.at[pl.ds(0, C)], didx1, sem_d1).wait()
    pltpu.make_async_copy(attr_hbm.at[pl.ds(0, C)], attrv1, sem_a1).wait()
    plsc.subcore_barrier()

    pltpu.sync_copy(acc_attr.at[pl.ds(r0, RPT)], out_attr.at[cid, pl.ds(r0, RPT)])
    pltpu.sync_copy(acc_deg.at[pl.ds(r0, RPT)], out_deg.at[cid, pl.ds(r0, RPT)])


def _sc_aggregate(featsS, src, dst, edge_attr, ones_d, zeros_d):
    mesh = plsc.VectorSubcoreMesh(core_axis_name="c", subcore_axis_name="s")
    node_fn = pl.kernel(
        _sc_node_body,
        mesh=mesh,
        compiler_params=pltpu.CompilerParams(use_tc_tiling_on_sc=False),
        out_type=[jax.ShapeDtypeStruct((NC, N_PAD, D_HALF), jnp.float32)],
        scratch_types=[
            pltpu.VMEM((C,), jnp.int32),
            pltpu.VMEM((C,), jnp.int32),
            pltpu.VMEM((C, D_HALF), jnp.float32),
            pltpu.VMEM((C,), jnp.int32),
            pltpu.VMEM((C,), jnp.int32),
            pltpu.VMEM((C, D_HALF), jnp.float32),
            pltpu.VMEM_SHARED((N_PAD, D_HALF), jnp.float32),
            pltpu.SemaphoreType.DMA,
            pltpu.SemaphoreType.DMA,
            pltpu.SemaphoreType.DMA,
            pltpu.SemaphoreType.DMA,
            pltpu.SemaphoreType.DMA,
            pltpu.SemaphoreType.DMA,
        ],
    )
    attr_fn = pl.kernel(
        _sc_attr_body,
        mesh=mesh,
        compiler_params=pltpu.CompilerParams(use_tc_tiling_on_sc=False),
        out_type=[
            jax.ShapeDtypeStruct((NC, N_PAD, D_EDGE), jnp.float32),
            jax.ShapeDtypeStruct((NC, N_PAD, D_DEG), jnp.float32),
        ],
        scratch_types=[
            pltpu.VMEM((C,), jnp.int32),
            pltpu.VMEM((C, D_EDGE), jnp.float32),
            pltpu.VMEM((C,), jnp.int32),
            pltpu.VMEM((C, D_EDGE), jnp.float32),
            pltpu.VMEM((C, D_DEG), jnp.float32),
            pltpu.VMEM_SHARED((N_PAD, D_EDGE), jnp.float32),
            pltpu.VMEM_SHARED((N_PAD, D_DEG), jnp.float32),
            pltpu.SemaphoreType.DMA,
            pltpu.SemaphoreType.DMA,
            pltpu.SemaphoreType.DMA,
            pltpu.SemaphoreType.DMA,
        ],
    )
    (node_p,) = node_fn(featsS, src, dst)
    attr_p, deg_p = attr_fn(dst, edge_attr, ones_d, zeros_d)
    return node_p, attr_p, deg_p


def _combine_body(np_ref, ap_ref, dp_ref, feats_ref,
                  wrelt_ref, wedget_ref, wrest_ref,
                  bcomb_ref, bres_ref, gamma_ref, beta_ref, out_ref):
    aggf = jnp.concatenate(
        [np_ref[0, :N_NODES, :], np_ref[1, :N_NODES, :]], axis=1)
    segattr = ap_ref[0, :N_NODES, :] + ap_ref[1, :N_NODES, :]
    deg = dp_ref[0, :N_NODES, 0:1] + dp_ref[1, :N_NODES, 0:1]
    agg = (jnp.dot(aggf, wrelt_ref[...], preferred_element_type=jnp.float32)
           + jnp.dot(segattr, wedget_ref[...], preferred_element_type=jnp.float32)
           + deg * bcomb_ref[...])
    new = jnp.maximum(agg, 0.0)
    res = jnp.maximum(
        jnp.dot(feats_ref[...], wrest_ref[...], preferred_element_type=jnp.float32)
        + bres_ref[...], 0.0)
    new = new + res
    mean = jnp.mean(new, axis=0, keepdims=True)
    var = jnp.mean((new - mean) ** 2, axis=0, keepdims=True)
    out_ref[...] = (new - mean) * lax.rsqrt(var + 1e-5) * gamma_ref[...] + beta_ref[...]


def _combine(node_p, attr_p, deg_p, feats, wrelt, wedget, wrest,
             bcomb, bres, gamma, beta):
    return pl.pallas_call(
        _combine_body,
        out_shape=jax.ShapeDtypeStruct((N_NODES, D_OUT), jnp.float32),
    )(node_p, attr_p, deg_p, feats, wrelt, wedget, wrest, bcomb, bres, gamma, beta)


def kernel(feats, edge_index, edge_attr, W_rel, b_rel, W_edge, b_edge,
           W_res, b_res, gamma, beta):
    src = edge_index[0]
    dst = edge_index[1]
    featsR = feats.reshape(2 * N_NODES, D_HALF)
    node_p, attr_p, deg_p = _sc_aggregate(
        featsR, src, dst, edge_attr,
        jnp.ones((C, D_DEG), jnp.float32), jnp.zeros((C, D_DEG), jnp.float32))
    return _combine(
        node_p, attr_p, deg_p, feats,
        W_rel.T, W_edge.T, W_res.T,
        (b_rel + b_edge).reshape(1, D_OUT), b_res.reshape(1, D_OUT),
        gamma.reshape(1, D_OUT), beta.reshape(1, D_OUT))


# speedup vs baseline: 1.9713x; 1.1830x over previous
import jax
import jax.numpy as jnp
from jax import lax
from jax.experimental import pallas as pl
from jax.experimental.pallas import tpu as pltpu
from jax.experimental.pallas import tpu_sc as plsc

N_NODES = 10000
N_PAD = 10112
D_IN = 128
D_OUT = 128
D_EDGE = 16
D_DEG = 8
D_HALF = 64
N_EDGES = 320000
C = 128
NCHUNKS = N_EDGES // C
NC = 2
NS = 16
NW = NC * NS
NSLOTS = 158
RPT = N_PAD // NS
ZCHUNKS = (128, 128, 128, 128, 120)
NSLOTS_A = 80


def _sc_node_body(featsS_hbm, src_hbm, dst_hbm,
                  out_node,
                  sidx0, didx0, rows0, sidx1, didx1, rows1, sidx2, didx2, rows2,
                  sidx3, didx3, rows3, sidx4, didx4, rows4,
                  acc_node,
                  sem_s0, sem_d0, sem_g0, sem_c0,
                  sem_s1, sem_d1, sem_g1, sem_c1,
                  sem_s2, sem_d2, sem_g2, sem_c2,
                  sem_s3, sem_d3, sem_g3, sem_c3,
                  sem_s4, sem_d4, sem_g4, sem_c4):
    cid = lax.axis_index("c")
    sid = lax.axis_index("s")
    zero16 = jnp.zeros((16,), jnp.float32)
    cid16 = lax.broadcast(cid, (16,)).astype(jnp.int32)
    dump16 = jnp.full((16,), N_NODES, jnp.int32)
    B = ((sidx0, didx0, rows0, sem_s0, sem_d0, sem_g0, sem_c0),
         (sidx1, didx1, rows1, sem_s1, sem_d1, sem_g1, sem_c1),
         (sidx2, didx2, rows2, sem_s2, sem_d2, sem_g2, sem_c2),
         (sidx3, didx3, rows3, sem_s3, sem_d3, sem_g3, sem_c3),
         (sidx4, didx4, rows4, sem_s4, sem_d4, sem_g4, sem_c4))

    def _fill_row(r, carry):
        for j in range(D_HALF // 16):
            rows0[r, pl.ds(j * 16, 16)] = zero16
        return carry
    lax.fori_loop(0, C, _fill_row, 0)

    r0 = sid * RPT
    zoff = 0
    for zc in ZCHUNKS:
        pltpu.sync_copy(rows0.at[pl.ds(0, zc)], acc_node.at[pl.ds(r0 + zoff, zc)])
        zoff += zc
    plsc.subcore_barrier()

    def _base(slot):
        return jnp.minimum(sid + NS * slot, NCHUNKS - 1) * C

    def _valid16(slot):
        v = (sid + NS * slot < NCHUNKS).astype(jnp.int32)
        return lax.broadcast(v, (16,))

    def _fix(slot, b):
        sidx, didx = b[0], b[1]
        v16 = _valid16(slot)
        for j in range(C // 16):
            sl = pl.ds(j * 16, 16)
            sidx[sl] = sidx[sl] * 2 + cid16
            didx[sl] = didx[sl] * v16 + dump16 * (1 - v16)

    def _start_loads(slot, b):
        pltpu.async_copy(src_hbm.at[pl.ds(_base(slot), C)], b[0], b[3])
        pltpu.async_copy(dst_hbm.at[pl.ds(_base(slot), C)], b[1], b[4])

    def _wait_loads(b):
        pltpu.make_async_copy(src_hbm.at[pl.ds(0, C)], b[0], b[3]).wait()
        pltpu.make_async_copy(dst_hbm.at[pl.ds(0, C)], b[1], b[4]).wait()

    def _start_gather(b):
        pltpu.async_copy(featsS_hbm.at[b[0]], b[2], b[5])

    def _wait_gather(b):
        pltpu.make_async_copy(featsS_hbm.at[b[0]], b[2], b[5]).wait()

    def _start_scat(b):
        pltpu.async_copy(b[2], acc_node.at[b[1]], b[6], add=True)

    def _wait_scat(b):
        pltpu.make_async_copy(b[2], acc_node.at[b[1]], b[6]).wait()

    def _slot(k, i, scat_wait):
        _wait_gather(B[i])
        _start_scat(B[i])
        if scat_wait:
            _wait_scat(B[(i - 2) % 5])
        _start_loads(k + 3, B[(i + 3) % 5])
        _wait_loads(B[(i + 2) % 5])
        _fix(k + 2, B[(i + 2) % 5])
        _start_gather(B[(i + 2) % 5])

    for slot in (0, 1):
        b = B[slot]
        pltpu.sync_copy(src_hbm.at[pl.ds(_base(slot), C)], b[0])
        pltpu.sync_copy(dst_hbm.at[pl.ds(_base(slot), C)], b[1])
        _fix(slot, b)
        _start_gather(b)
    _start_loads(2, B[2])

    _slot(0, 0, scat_wait=False)
    _slot(1, 1, scat_wait=False)

    def _group(m, carry):
        k0 = 2 + 5 * m
        for i in range(5):
            _slot(k0 + i, (2 + i) % 5, scat_wait=True)
        return carry
    lax.fori_loop(0, (NSLOTS - 3) // 5, _group, 0)

    k = NSLOTS - 1
    _wait_gather(B[k % 5])
    _start_scat(B[k % 5])
    _wait_scat(B[(k - 2) % 5])
    _wait_scat(B[(k - 1) % 5])
    _wait_scat(B[k % 5])
    _wait_gather(B[(k + 1) % 5])
    _wait_loads(B[(k + 2) % 5])
    plsc.subcore_barrier()

    pltpu.sync_copy(acc_node.at[pl.ds(r0, RPT)], out_node.at[cid, pl.ds(r0, RPT)])


def _sc_attr_body(dst_hbm, attr_hbm, ones_hbm, zeros_hbm,
                  out_attr, out_deg,
                  didx0, attrv0, didx1, attrv1, ones_v,
                  acc_attr, acc_deg,
                  sem_d0, sem_a0, sem_d1, sem_a1):
    cid = lax.axis_index("c")
    sid = lax.axis_index("s")
    wid = sid * NC + cid
    zero16 = jnp.zeros((16,), jnp.float32)
    dump16 = jnp.full((16,), N_NODES, jnp.int32)

    def _fill_row(r, carry):
        attrv0[r, :] = zero16
        return carry
    lax.fori_loop(0, C, _fill_row, 0)
    pltpu.sync_copy(ones_hbm, ones_v)

    r0 = sid * RPT
    zoff = 0
    for zc in ZCHUNKS:
        pltpu.sync_copy(attrv0.at[pl.ds(0, zc)], acc_attr.at[pl.ds(r0 + zoff, zc)])
        pltpu.sync_copy(zeros_hbm.at[pl.ds(0, zc)], acc_deg.at[pl.ds(r0 + zoff, zc)])
        zoff += zc
    plsc.subcore_barrier()

    def _base(slot):
        return jnp.minimum(wid + NW * slot, NCHUNKS - 1) * C

    def _valid16(slot):
        v = (wid + NW * slot < NCHUNKS).astype(jnp.int32)
        return lax.broadcast(v, (16,))

    def _mask(didx, v16):
        for j in range(C // 16):
            sl = pl.ds(j * 16, 16)
            didx[sl] = didx[sl] * v16 + dump16 * (1 - v16)

    def _start_loads(slot, didx, attrv, sd, sa):
        b = _base(slot)
        pltpu.async_copy(dst_hbm.at[pl.ds(b, C)], didx, sd)
        pltpu.async_copy(attr_hbm.at[pl.ds(b, C)], attrv, sa)

    def _wait_loads(didx, attrv, sd, sa):
        pltpu.make_async_copy(dst_hbm.at[pl.ds(0, C)], didx, sd).wait()
        pltpu.make_async_copy(attr_hbm.at[pl.ds(0, C)], attrv, sa).wait()

    b0 = _base(0)
    pltpu.sync_copy(dst_hbm.at[pl.ds(b0, C)], didx0)
    pltpu.sync_copy(attr_hbm.at[pl.ds(b0, C)], attrv0)
    _mask(didx0, _valid16(0))
    _start_loads(1, didx1, attrv1, sem_d1, sem_a1)

    def _pair(p, carry):
        pltpu.sync_copy(attrv0, acc_attr.at[didx0], add=True)
        pltpu.sync_copy(ones_v, acc_deg.at[didx0], add=True)
        _wait_loads(didx1, attrv1, sem_d1, sem_a1)
        _mask(didx1, _valid16(2 * p + 1))
        _start_loads(2 * p + 2, didx0, attrv0, sem_d0, sem_a0)
        pltpu.sync_copy(attrv1, acc_attr.at[didx1], add=True)
        pltpu.sync_copy(ones_v, acc_deg.at[didx1], add=True)
        _start_loads(2 * p + 3, didx1, attrv1, sem_d1, sem_a1)
        _wait_loads(didx0, attrv0, sem_d0, sem_a0)
        _mask(didx0, _valid16(2 * p + 2))
        return carry
    lax.fori_loop(0, NSLOTS_A // 2, _pair, 0)

    pltpu.make_async_copy(dst_hbm.at[pl.ds(0, C)], didx1, sem_d1).wait()
    pltpu.make_async_copy(attr_hbm.at[pl.ds(0, C)], attrv1, sem_a1).wait()
    plsc.subcore_barrier()

    pltpu.sync_copy(acc_attr.at[pl.ds(r0, RPT)], out_attr.at[cid, pl.ds(r0, RPT)])
    pltpu.sync_copy(acc_deg.at[pl.ds(r0, RPT)], out_deg.at[cid, pl.ds(r0, RPT)])


def _sc_aggregate(featsS, src, dst, edge_attr, ones_d, zeros_d):
    mesh = plsc.VectorSubcoreMesh(core_axis_name="c", subcore_axis_name="s")
    node_fn = pl.kernel(
        _sc_node_body,
        mesh=mesh,
        compiler_params=pltpu.CompilerParams(use_tc_tiling_on_sc=False),
        out_type=[jax.ShapeDtypeStruct((NC, N_PAD, D_HALF), jnp.float32)],
        scratch_types=(
            [pltpu.VMEM((C,), jnp.int32),
             pltpu.VMEM((C,), jnp.int32),
             pltpu.VMEM((C, D_HALF), jnp.float32)] * 5
            + [pltpu.VMEM_SHARED((N_PAD, D_HALF), jnp.float32)]
            + [pltpu.SemaphoreType.DMA] * 20
        ),
    )
    attr_fn = pl.kernel(
        _sc_attr_body,
        mesh=mesh,
        compiler_params=pltpu.CompilerParams(use_tc_tiling_on_sc=False),
        out_type=[
            jax.ShapeDtypeStruct((NC, N_PAD, D_EDGE), jnp.float32),
            jax.ShapeDtypeStruct((NC, N_PAD, D_DEG), jnp.float32),
        ],
        scratch_types=[
            pltpu.VMEM((C,), jnp.int32),
            pltpu.VMEM((C, D_EDGE), jnp.float32),
            pltpu.VMEM((C,), jnp.int32),
            pltpu.VMEM((C, D_EDGE), jnp.float32),
            pltpu.VMEM((C, D_DEG), jnp.float32),
            pltpu.VMEM_SHARED((N_PAD, D_EDGE), jnp.float32),
            pltpu.VMEM_SHARED((N_PAD, D_DEG), jnp.float32),
            pltpu.SemaphoreType.DMA,
            pltpu.SemaphoreType.DMA,
            pltpu.SemaphoreType.DMA,
            pltpu.SemaphoreType.DMA,
        ],
    )
    (node_p,) = node_fn(featsS, src, dst)
    attr_p, deg_p = attr_fn(dst, edge_attr, ones_d, zeros_d)
    return node_p, attr_p, deg_p


def _combine_body(np_ref, ap_ref, dp_ref, feats_ref,
                  wrelt_ref, wedget_ref, wrest_ref,
                  bcomb_ref, bres_ref, gamma_ref, beta_ref, out_ref):
    aggf = jnp.concatenate(
        [np_ref[0, :N_NODES, :], np_ref[1, :N_NODES, :]], axis=1)
    segattr = ap_ref[0, :N_NODES, :] + ap_ref[1, :N_NODES, :]
    deg = dp_ref[0, :N_NODES, 0:1] + dp_ref[1, :N_NODES, 0:1]
    agg = (jnp.dot(aggf, wrelt_ref[...], preferred_element_type=jnp.float32)
           + jnp.dot(segattr, wedget_ref[...], preferred_element_type=jnp.float32)
           + deg * bcomb_ref[...])
    new = jnp.maximum(agg, 0.0)
    res = jnp.maximum(
        jnp.dot(feats_ref[...], wrest_ref[...], preferred_element_type=jnp.float32)
        + bres_ref[...], 0.0)
    new = new + res
    mean = jnp.mean(new, axis=0, keepdims=True)
    var = jnp.mean((new - mean) ** 2, axis=0, keepdims=True)
    out_ref[...] = (new - mean) * lax.rsqrt(var + 1e-5) * gamma_ref[...] + beta_ref[...]


def _combine(node_p, attr_p, deg_p, feats, wrelt, wedget, wrest,
             bcomb, bres, gamma, beta):
    return pl.pallas_call(
        _combine_body,
        out_shape=jax.ShapeDtypeStruct((N_NODES, D_OUT), jnp.float32),
    )(node_p, attr_p, deg_p, feats, wrelt, wedget, wrest, bcomb, bres, gamma, beta)


def kernel(feats, edge_index, edge_attr, W_rel, b_rel, W_edge, b_edge,
           W_res, b_res, gamma, beta):
    src = edge_index[0]
    dst = edge_index[1]
    featsR = feats.reshape(2 * N_NODES, D_HALF)
    node_p, attr_p, deg_p = _sc_aggregate(
        featsR, src, dst, edge_attr,
        jnp.ones((C, D_DEG), jnp.float32), jnp.zeros((C, D_DEG), jnp.float32))
    return _combine(
        node_p, attr_p, deg_p, feats,
        W_rel.T, W_edge.T, W_res.T,
        (b_rel + b_edge).reshape(1, D_OUT), b_res.reshape(1, D_OUT),
        gamma.reshape(1, D_OUT), beta.reshape(1, D_OUT))


# speedup vs baseline: 2.0169x; 1.0231x over previous
import jax
import jax.numpy as jnp
from jax import lax
from jax.experimental import pallas as pl
from jax.experimental.pallas import tpu as pltpu
from jax.experimental.pallas import tpu_sc as plsc

N_NODES = 10000
N_PAD = 10112
D_IN = 128
D_OUT = 128
D_EDGE = 16
D_DEG = 8
D_HALF = 64
N_EDGES = 320000
C = 128
NCHUNKS = N_EDGES // C
NC = 2
NS = 16
NW = NC * NS
NSLOTS = 158
RPT = N_PAD // NS
ZCHUNKS = (128, 128, 128, 128, 120)
NSLOTS_A = 82


def _sc_node_body(featsS_hbm, src_hbm, dst_hbm,
                  out_node,
                  sidx0, didx0, rows0, sidx1, didx1, rows1, sidx2, didx2, rows2,
                  sidx3, didx3, rows3, sidx4, didx4, rows4,
                  acc_node,
                  sem_s0, sem_d0, sem_g0, sem_c0,
                  sem_s1, sem_d1, sem_g1, sem_c1,
                  sem_s2, sem_d2, sem_g2, sem_c2,
                  sem_s3, sem_d3, sem_g3, sem_c3,
                  sem_s4, sem_d4, sem_g4, sem_c4):
    cid = lax.axis_index("c")
    sid = lax.axis_index("s")
    zero16 = jnp.zeros((16,), jnp.float32)
    cid16 = lax.broadcast(cid, (16,)).astype(jnp.int32)
    dump16 = jnp.full((16,), N_NODES, jnp.int32)
    B = ((sidx0, didx0, rows0, sem_s0, sem_d0, sem_g0, sem_c0),
         (sidx1, didx1, rows1, sem_s1, sem_d1, sem_g1, sem_c1),
         (sidx2, didx2, rows2, sem_s2, sem_d2, sem_g2, sem_c2),
         (sidx3, didx3, rows3, sem_s3, sem_d3, sem_g3, sem_c3),
         (sidx4, didx4, rows4, sem_s4, sem_d4, sem_g4, sem_c4))

    def _fill_row(r, carry):
        for j in range(D_HALF // 16):
            rows0[r, pl.ds(j * 16, 16)] = zero16
        return carry
    lax.fori_loop(0, C, _fill_row, 0)

    r0 = sid * RPT
    zoff = 0
    for zc in ZCHUNKS:
        pltpu.sync_copy(rows0.at[pl.ds(0, zc)], acc_node.at[pl.ds(r0 + zoff, zc)])
        zoff += zc
    plsc.subcore_barrier()

    def _base(slot):
        return jnp.minimum(sid + NS * slot, NCHUNKS - 1) * C

    def _valid16(slot):
        v = (sid + NS * slot < NCHUNKS).astype(jnp.int32)
        return lax.broadcast(v, (16,))

    def _fix(slot, b):
        sidx, didx = b[0], b[1]
        v16 = _valid16(slot)
        for j in range(C // 16):
            sl = pl.ds(j * 16, 16)
            sidx[sl] = sidx[sl] * 2 + cid16
            didx[sl] = didx[sl] * v16 + dump16 * (1 - v16)

    def _start_loads(slot, b):
        pltpu.async_copy(src_hbm.at[pl.ds(_base(slot), C)], b[0], b[3])
        pltpu.async_copy(dst_hbm.at[pl.ds(_base(slot), C)], b[1], b[4])

    def _wait_loads(b):
        pltpu.make_async_copy(src_hbm.at[pl.ds(0, C)], b[0], b[3]).wait()
        pltpu.make_async_copy(dst_hbm.at[pl.ds(0, C)], b[1], b[4]).wait()

    def _start_gather(b):
        pltpu.async_copy(featsS_hbm.at[b[0]], b[2], b[5])

    def _wait_gather(b):
        pltpu.make_async_copy(featsS_hbm.at[b[0]], b[2], b[5]).wait()

    def _start_scat(b):
        pltpu.async_copy(b[2], acc_node.at[b[1]], b[6], add=True)

    def _wait_scat(b):
        pltpu.make_async_copy(b[2], acc_node.at[b[1]], b[6]).wait()

    def _slot(k, i, scat_wait):
        _wait_gather(B[i])
        _start_scat(B[i])
        if scat_wait:
            _wait_scat(B[(i - 2) % 5])
        _start_loads(k + 3, B[(i + 3) % 5])
        _wait_loads(B[(i + 2) % 5])
        _fix(k + 2, B[(i + 2) % 5])
        _start_gather(B[(i + 2) % 5])

    for slot in (0, 1):
        b = B[slot]
        pltpu.sync_copy(src_hbm.at[pl.ds(_base(slot), C)], b[0])
        pltpu.sync_copy(dst_hbm.at[pl.ds(_base(slot), C)], b[1])
        _fix(slot, b)
        _start_gather(b)
    _start_loads(2, B[2])

    _slot(0, 0, scat_wait=False)
    _slot(1, 1, scat_wait=False)

    def _group(m, carry):
        k0 = 2 + 5 * m
        for i in range(5):
            _slot(k0 + i, (2 + i) % 5, scat_wait=True)
        return carry
    lax.fori_loop(0, (NSLOTS - 3) // 5, _group, 0)

    k = NSLOTS - 1
    _wait_gather(B[k % 5])
    _start_scat(B[k % 5])
    _wait_scat(B[(k - 2) % 5])
    _wait_scat(B[(k - 1) % 5])
    _wait_scat(B[k % 5])
    _wait_gather(B[(k + 1) % 5])
    _wait_loads(B[(k + 2) % 5])
    plsc.subcore_barrier()

    pltpu.sync_copy(acc_node.at[pl.ds(r0, RPT)], out_node.at[cid, pl.ds(r0, RPT)])


def _sc_attr_body(dst_hbm, attr_hbm, ones_hbm, zeros_hbm,
                  out_attr, out_deg,
                  didx0, attrv0, didx1, attrv1, didx2, attrv2, didx3, attrv3,
                  ones_v,
                  acc_attr, acc_deg,
                  sem_d0, sem_a0, sem_ca0, sem_cd0,
                  sem_d1, sem_a1, sem_ca1, sem_cd1,
                  sem_d2, sem_a2, sem_ca2, sem_cd2,
                  sem_d3, sem_a3, sem_ca3, sem_cd3):
    cid = lax.axis_index("c")
    sid = lax.axis_index("s")
    wid = sid * NC + cid
    zero16 = jnp.zeros((16,), jnp.float32)
    dump16 = jnp.full((16,), N_NODES, jnp.int32)
    B = ((didx0, attrv0, sem_d0, sem_a0, sem_ca0, sem_cd0),
         (didx1, attrv1, sem_d1, sem_a1, sem_ca1, sem_cd1),
         (didx2, attrv2, sem_d2, sem_a2, sem_ca2, sem_cd2),
         (didx3, attrv3, sem_d3, sem_a3, sem_ca3, sem_cd3))

    def _fill_row(r, carry):
        attrv0[r, :] = zero16
        return carry
    lax.fori_loop(0, C, _fill_row, 0)
    pltpu.sync_copy(ones_hbm, ones_v)

    r0 = sid * RPT
    zoff = 0
    for zc in ZCHUNKS:
        pltpu.sync_copy(attrv0.at[pl.ds(0, zc)], acc_attr.at[pl.ds(r0 + zoff, zc)])
        pltpu.sync_copy(zeros_hbm.at[pl.ds(0, zc)], acc_deg.at[pl.ds(r0 + zoff, zc)])
        zoff += zc
    plsc.subcore_barrier()

    def _base(slot):
        return jnp.minimum(wid + NW * slot, NCHUNKS - 1) * C

    def _mask(slot, didx):
        v16 = lax.broadcast((wid + NW * slot < NCHUNKS).astype(jnp.int32), (16,))
        for j in range(C // 16):
            sl = pl.ds(j * 16, 16)
            didx[sl] = didx[sl] * v16 + dump16 * (1 - v16)

    def _start_loads(slot, b):
        pltpu.async_copy(dst_hbm.at[pl.ds(_base(slot), C)], b[0], b[2])
        pltpu.async_copy(attr_hbm.at[pl.ds(_base(slot), C)], b[1], b[3])

    def _wait_loads(b):
        pltpu.make_async_copy(dst_hbm.at[pl.ds(0, C)], b[0], b[2]).wait()
        pltpu.make_async_copy(attr_hbm.at[pl.ds(0, C)], b[1], b[3]).wait()

    def _start_scats(b):
        pltpu.async_copy(b[1], acc_attr.at[b[0]], b[4], add=True)
        pltpu.async_copy(ones_v, acc_deg.at[b[0]], b[5], add=True)

    def _wait_scats(b):
        pltpu.make_async_copy(b[1], acc_attr.at[b[0]], b[4]).wait()
        pltpu.make_async_copy(ones_v, acc_deg.at[b[0]], b[5]).wait()

    def _slot(k, i, scat_wait):
        if scat_wait:
            _wait_scats(B[(i + 2) % 4])
        _start_loads(k + 2, B[(i + 2) % 4])
        _start_scats(B[i])
        _wait_loads(B[(i + 1) % 4])
        _mask(k + 1, B[(i + 1) % 4][0])

    pltpu.sync_copy(dst_hbm.at[pl.ds(_base(0), C)], didx0)
    pltpu.sync_copy(attr_hbm.at[pl.ds(_base(0), C)], attrv0)
    _mask(0, didx0)
    _start_loads(1, B[1])

    _slot(0, 0, scat_wait=False)
    _slot(1, 1, scat_wait=False)

    def _group(m, carry):
        k0 = 2 + 4 * m
        for i in range(4):
            _slot(k0 + i, (2 + i) % 4, scat_wait=True)
        return carry
    lax.fori_loop(0, (NSLOTS_A - 2) // 4, _group, 0)

    _wait_scats(B[(NSLOTS_A - 2) % 4])
    _wait_scats(B[(NSLOTS_A - 1) % 4])
    _wait_loads(B[(NSLOTS_A + 1) % 4])
    plsc.subcore_barrier()

    pltpu.sync_copy(acc_attr.at[pl.ds(r0, RPT)], out_attr.at[cid, pl.ds(r0, RPT)])
    pltpu.sync_copy(acc_deg.at[pl.ds(r0, RPT)], out_deg.at[cid, pl.ds(r0, RPT)])


def _sc_aggregate(featsS, src, dst, edge_attr, ones_d, zeros_d):
    mesh = plsc.VectorSubcoreMesh(core_axis_name="c", subcore_axis_name="s")
    node_fn = pl.kernel(
        _sc_node_body,
        mesh=mesh,
        compiler_params=pltpu.CompilerParams(use_tc_tiling_on_sc=False),
        out_type=[jax.ShapeDtypeStruct((NC, N_PAD, D_HALF), jnp.float32)],
        scratch_types=(
            [pltpu.VMEM((C,), jnp.int32),
             pltpu.VMEM((C,), jnp.int32),
             pltpu.VMEM((C, D_HALF), jnp.float32)] * 5
            + [pltpu.VMEM_SHARED((N_PAD, D_HALF), jnp.float32)]
            + [pltpu.SemaphoreType.DMA] * 20
        ),
    )
    attr_fn = pl.kernel(
        _sc_attr_body,
        mesh=mesh,
        compiler_params=pltpu.CompilerParams(use_tc_tiling_on_sc=False),
        out_type=[
            jax.ShapeDtypeStruct((NC, N_PAD, D_EDGE), jnp.float32),
            jax.ShapeDtypeStruct((NC, N_PAD, D_DEG), jnp.float32),
        ],
        scratch_types=(
            [pltpu.VMEM((C,), jnp.int32),
             pltpu.VMEM((C, D_EDGE), jnp.float32)] * 4
            + [pltpu.VMEM((C, D_DEG), jnp.float32),
               pltpu.VMEM_SHARED((N_PAD, D_EDGE), jnp.float32),
               pltpu.VMEM_SHARED((N_PAD, D_DEG), jnp.float32)]
            + [pltpu.SemaphoreType.DMA] * 16
        ),
    )
    (node_p,) = node_fn(featsS, src, dst)
    attr_p, deg_p = attr_fn(dst, edge_attr, ones_d, zeros_d)
    return node_p, attr_p, deg_p


def _combine_body(np_ref, ap_ref, dp_ref, feats_ref,
                  wrelt_ref, wedget_ref, wrest_ref,
                  bcomb_ref, bres_ref, gamma_ref, beta_ref, out_ref):
    aggf = jnp.concatenate(
        [np_ref[0, :N_NODES, :], np_ref[1, :N_NODES, :]], axis=1)
    segattr = ap_ref[0, :N_NODES, :] + ap_ref[1, :N_NODES, :]
    deg = dp_ref[0, :N_NODES, 0:1] + dp_ref[1, :N_NODES, 0:1]
    agg = (jnp.dot(aggf, wrelt_ref[...], preferred_element_type=jnp.float32)
           + jnp.dot(segattr, wedget_ref[...], preferred_element_type=jnp.float32)
           + deg * bcomb_ref[...])
    new = jnp.maximum(agg, 0.0)
    res = jnp.maximum(
        jnp.dot(feats_ref[...], wrest_ref[...], preferred_element_type=jnp.float32)
        + bres_ref[...], 0.0)
    new = new + res
    mean = jnp.mean(new, axis=0, keepdims=True)
    var = jnp.mean((new - mean) ** 2, axis=0, keepdims=True)
    out_ref[...] = (new - mean) * lax.rsqrt(var + 1e-5) * gamma_ref[...] + beta_ref[...]


def _combine(node_p, attr_p, deg_p, feats, wrelt, wedget, wrest,
             bcomb, bres, gamma, beta):
    return pl.pallas_call(
        _combine_body,
        out_shape=jax.ShapeDtypeStruct((N_NODES, D_OUT), jnp.float32),
    )(node_p, attr_p, deg_p, feats, wrelt, wedget, wrest, bcomb, bres, gamma, beta)


def kernel(feats, edge_index, edge_attr, W_rel, b_rel, W_edge, b_edge,
           W_res, b_res, gamma, beta):
    src = edge_index[0]
    dst = edge_index[1]
    featsR = feats.reshape(2 * N_NODES, D_HALF)
    node_p, attr_p, deg_p = _sc_aggregate(
        featsR, src, dst, edge_attr,
        jnp.ones((C, D_DEG), jnp.float32), jnp.zeros((C, D_DEG), jnp.float32))
    return _combine(
        node_p, attr_p, deg_p, feats,
        W_rel.T, W_edge.T, W_res.T,
        (b_rel + b_edge).reshape(1, D_OUT), b_res.reshape(1, D_OUT),
        gamma.reshape(1, D_OUT), beta.reshape(1, D_OUT))
